# baseline (speedup 1.0000x reference)
import functools

import numpy as np
import jax
import jax.numpy as jnp
from jax import lax
from jax.experimental import pallas as pl
from jax.experimental.pallas import tpu as pltpu
from jax.experimental.pallas import tpu_sc as plsc

N_LEVELS = 16
F_ENC = 8
T_SIZE = 1 << 19
BASE_RES = 16
GROWTH = 1.4472692012786865
N_COND = 4

NC, NS = 2, 16
NW = NC * NS
GRP = 16
NIDX = GRP * 8

K1 = -1640531535
K2 = 805459861
MASK = T_SIZE - 1

_LEVELS = []
for _l in range(N_LEVELS):
    _scale = BASE_RES * (GROWTH ** _l) - 1.0
    _res = int(np.ceil(_scale)) + 1
    _use_hash = (_res + 1) ** 3 > T_SIZE
    _LEVELS.append((_scale, _use_hash, _res + 1))


def _sc_body(chunk, table_ref, in_ref, code_ref, out_ref,
             xyz_v, code_v, out_v, idx_v, rows_v, sem0, sem1):
    ngroups = chunk // GRP
    wid = lax.axis_index("s") * NC + lax.axis_index("c")
    pltpu.sync_copy(in_ref.at[wid], xyz_v)
    pltpu.sync_copy(code_ref.at[wid], code_v)

    iota = lax.iota(jnp.int32, GRP)
    row_ids = [iota + c * GRP for c in range(8)]
    col_ids = [jnp.full((GRP,), j, jnp.int32) for j in range(8)]
    sems = (sem0, sem1)

    for l, (scale, use_hash, stride) in enumerate(_LEVELS):
        base_l = l * T_SIZE

        def idx_for(g, buf, scale=scale, use_hash=use_hash,
                    stride=stride, base_l=base_l):
            s0 = g * GRP
            x = xyz_v[0, pl.ds(s0, GRP)]
            y = xyz_v[1, pl.ds(s0, GRP)]
            z = xyz_v[2, pl.ds(s0, GRP)]
            px = (x * scale + 0.5).astype(jnp.int32)
            py = (y * scale + 0.5).astype(jnp.int32)
            pz = (z * scale + 0.5).astype(jnp.int32)
            if use_hash:
                h1 = py * K1
                h2 = pz * K2
                px1 = px + 1
                h1b = h1 + K1
                h2b = h2 + K2
                for c in range(8):
                    a = px1 if (c & 1) else px
                    hy = h1b if ((c >> 1) & 1) else h1
                    hz = h2b if ((c >> 2) & 1) else h2
                    idx = ((a ^ hy) ^ hz) & MASK
                    idx_v[buf, pl.ds(c * GRP, GRP)] = idx + base_l
            else:
                s2 = stride * stride
                basei = px + py * stride + pz * s2 + base_l
                for c in range(8):
                    off = (c & 1) + ((c >> 1) & 1) * stride + ((c >> 2) & 1) * s2
                    idx_v[buf, pl.ds(c * GRP, GRP)] = basei + off

        def fire(buf):
            pltpu.async_copy(table_ref.at[idx_v.at[buf]], rows_v.at[buf],
                             sems[buf])

        def blend(g, buf, l=l, scale=scale):
            s0 = g * GRP
            x = xyz_v[0, pl.ds(s0, GRP)]
            y = xyz_v[1, pl.ds(s0, GRP)]
            z = xyz_v[2, pl.ds(s0, GRP)]
            posx = x * scale + 0.5
            posy = y * scale + 0.5
            posz = z * scale + 0.5
            fx = posx - posx.astype(jnp.int32).astype(jnp.float32)
            fy = posy - posy.astype(jnp.int32).astype(jnp.float32)
            fz = posz - posz.astype(jnp.int32).astype(jnp.float32)
            wxs = (1.0 - fx, fx)
            wys = (1.0 - fy, fy)
            wzs = (1.0 - fz, fz)
            rows_b = rows_v.at[buf]
            feats = [None] * 8
            for c in range(8):
                w = wxs[c & 1] * wys[(c >> 1) & 1] * wzs[(c >> 2) & 1]
                for j in range(8):
                    gval = plsc.load_gather(rows_b, [row_ids[c], col_ids[j]])
                    t = w * gval
                    feats[j] = t if c == 0 else feats[j] + t
            c0 = code_v[0, pl.ds(s0, GRP)]
            c1 = code_v[1, pl.ds(s0, GRP)]
            c2 = code_v[2, pl.ds(s0, GRP)]
            c3 = code_v[3, pl.ds(s0, GRP)]
            out_v[2 * l, pl.ds(s0, GRP)] = (
                c0 * feats[0] + c1 * feats[2] + c2 * feats[4] + c3 * feats[6])
            out_v[2 * l + 1, pl.ds(s0, GRP)] = (
                c0 * feats[1] + c1 * feats[3] + c2 * feats[5] + c3 * feats[7])

        idx_for(0, 0)
        fire(0)

        def outer(it, carry):
            for bpar in range(2):
                g = it * 2 + bpar
                nxt = g + 1

                @pl.when(nxt < ngroups)
                def _():
                    idx_for(nxt, 1 - bpar)
                    fire(1 - bpar)

                pltpu.make_async_copy(table_ref.at[idx_v.at[bpar]],
                                      rows_v.at[bpar], sems[bpar]).wait()
                blend(g, bpar)
            return carry

        lax.fori_loop(0, ngroups // 2, outer, jnp.int32(0))

    pltpu.sync_copy(out_v, out_ref.at[wid])


@jax.jit
def kernel(in_tensor, conditioning_code, hash_table):
    b = in_tensor.shape[0]
    chunk = b // NW
    table_flat = hash_table.reshape(N_LEVELS * T_SIZE, F_ENC)
    in_arr = in_tensor.T.reshape(3, NW, chunk).transpose(1, 0, 2)
    code_arr = conditioning_code.T.reshape(N_COND, NW, chunk).transpose(1, 0, 2)

    mesh = plsc.VectorSubcoreMesh(core_axis_name="c", subcore_axis_name="s")
    fn = pl.kernel(
        functools.partial(_sc_body, chunk),
        mesh=mesh,
        compiler_params=pltpu.CompilerParams(
            needs_layout_passes=False, use_tc_tiling_on_sc=False),
        out_type=jax.ShapeDtypeStruct((NW, 2 * N_LEVELS, chunk), jnp.float32),
        scratch_types=[
            pltpu.VMEM((3, chunk), jnp.float32),
            pltpu.VMEM((N_COND, chunk), jnp.float32),
            pltpu.VMEM((2 * N_LEVELS, chunk), jnp.float32),
            pltpu.VMEM((2, NIDX), jnp.int32),
            pltpu.VMEM((2, NIDX, F_ENC), jnp.float32),
            pltpu.SemaphoreType.DMA,
            pltpu.SemaphoreType.DMA,
        ],
    )
    out = fn(table_flat, in_arr, code_arr)
    return out.transpose(0, 2, 1).reshape(b, 2 * N_LEVELS)
